# Initial kernel scaffold; baseline (speedup 1.0000x reference)
import jax
import jax.numpy as jnp
from jax.experimental import pallas as pl


def kernel(locations, box_cls, box_regression, centerness, size):
    raise NotImplementedError("write your pallas kernel here")


# speedup vs baseline: 10.5854x; 10.5854x over previous
import functools

import jax
import jax.numpy as jnp
from jax import lax
from jax.experimental import pallas as pl
from jax.experimental.pallas import tpu as pltpu

N, C, H, W = 4, 80, 160, 160
HW = H * W
HB = 10
Hb = H // HB
BLK = Hb * W
K = 100
THR = 0.2
NMS_THR = 0.6
PAD = 128
BIG = 1 << 30

_DOT = dict(precision=lax.Precision.HIGHEST, preferred_element_type=jnp.float32)


def _sigmoid(x):
    return 1.0 / (1.0 + jnp.exp(-x))


def _masked_scores(cls_blk, ctr_blk):
    sig = _sigmoid(cls_blk.reshape(C, BLK))
    ctr = _sigmoid(ctr_blk.reshape(1, BLK))
    return jnp.where(sig > THR, sig * ctr, -1.0)


def _k1_body(cls_ref, ctr_ref, locmax_ref):
    m = _masked_scores(cls_ref[0], ctr_ref[0, 0])
    locmax_ref[0, 0, :] = jnp.max(m, axis=0)


def _k2_body(locmax_ref, selhw_ref):
    x = locmax_ref[:]
    hw_iota = lax.broadcasted_iota(jnp.int32, (N, HW), 1)
    lane = lax.broadcasted_iota(jnp.int32, (N, PAD), 1)

    def body(i, carry):
        x, sel = carry
        m = jnp.max(x, axis=1, keepdims=True)
        win = jnp.min(jnp.where(x == m, hw_iota, BIG), axis=1, keepdims=True)
        sel = jnp.where(lane == i, win, sel)
        x = jnp.where(hw_iota == win, -2.0, x)
        return x, sel

    _, sel = lax.fori_loop(0, K, body, (x, jnp.full((N, PAD), -1, jnp.int32)))
    selhw_ref[:] = sel


def _k3_body(cls_ref, ctr_ref, reg_ref, loc_ref, selhw_ref,
             sco_ref, reg_out_ref, loc_out_ref):
    j = pl.program_id(1)
    m = _masked_scores(cls_ref[0], ctr_ref[0, 0])
    sel = jnp.reshape(selhw_ref[0, 0, :], (PAD, 1))
    hw_ids = j * BLK + lax.broadcasted_iota(jnp.int32, (PAD, BLK), 1)
    oh = (sel == hw_ids).astype(jnp.float32)

    sco = lax.dot_general(m, oh, (((1,), (1,)), ((), ())), **_DOT)
    reg2 = reg_ref[0].reshape(4, BLK)
    reg = lax.dot_general(reg2, oh, (((1,), (1,)), ((), ())), **_DOT)
    locT = lax.dot_general(loc_ref[:], oh, (((0,), (1,)), ((), ())), **_DOT)

    @pl.when(j == 0)
    def _():
        sco_ref[0] = jnp.zeros_like(sco_ref[0])
        reg_out_ref[0] = jnp.zeros_like(reg_out_ref[0])
        loc_out_ref[0] = jnp.zeros_like(loc_out_ref[0])

    sco_ref[0] += sco
    reg_out_ref[0] += reg
    loc_out_ref[0] += locT


def _k4_body(sco_ref, selhw_ref, reg_ref, loc_ref, size_ref,
             boxes_ref, scores_ref, labels_ref, keep_ref, iou_s):
    x = sco_ref[:]
    selhw = selhw_ref[:]
    c_iota = lax.broadcasted_iota(jnp.int32, (N, C, PAD), 1)
    r_iota = lax.broadcasted_iota(jnp.int32, (N, C, PAD), 2)
    elem_idx = selhw[:, None, :] * C + c_iota
    lane = lax.broadcasted_iota(jnp.int32, (N, PAD), 1)

    padmask = selhw[:, None, :] < 0
    x = jnp.where(padmask, -jnp.inf, x)

    def extract(i, carry):
        x, topv, topr, topc = carry
        m1 = jnp.max(x, axis=1)
        m = jnp.max(m1, axis=1)
        mb = m[:, None, None]
        cand = x == mb
        wi = jnp.min(jnp.min(jnp.where(cand, elem_idx, BIG), axis=1), axis=1)
        wib = wi[:, None, None]
        hit = cand & (elem_idx == wib)
        wr = jnp.min(jnp.min(jnp.where(hit, r_iota, BIG), axis=1), axis=1)
        wc = wi % C
        eq = lane == i
        topv = jnp.where(eq, m[:, None], topv)
        topr = jnp.where(eq, wr[:, None], topr)
        topc = jnp.where(eq, wc[:, None], topc)
        x = jnp.where(elem_idx == wib, -jnp.inf, x)
        return x, topv, topr, topc

    init = (x, jnp.zeros((N, PAD), jnp.float32),
            jnp.zeros((N, PAD), jnp.int32), jnp.zeros((N, PAD), jnp.int32))
    _, topv, topr, topc = lax.fori_loop(0, K, extract, init)

    slot_rows = []
    for n in range(N):
        ohn = (topr[n][:, None] == lax.broadcasted_iota(jnp.int32, (PAD, PAD), 1)).astype(jnp.float32)
        data = jnp.concatenate([reg_ref[n], loc_ref[n]], axis=0)
        slot_rows.append(lax.dot_general(data, ohn, (((1,), (1,)), ((), ())), **_DOT))
    sd = jnp.stack(slot_rows, axis=0)

    lx, ly = sd[:, 4, :], sd[:, 5, :]
    x1 = lx - sd[:, 0, :]
    y1 = ly - sd[:, 1, :]
    x2 = lx + sd[:, 2, :]
    y2 = ly + sd[:, 3, :]
    h_img = size_ref[0, 0].astype(jnp.float32)
    w_img = size_ref[0, 1].astype(jnp.float32)
    x1 = jnp.clip(x1, 0.0, w_img - 1.0)
    x2 = jnp.clip(x2, 0.0, w_img - 1.0)
    y1 = jnp.clip(y1, 0.0, h_img - 1.0)
    y2 = jnp.clip(y2, 0.0, h_img - 1.0)

    valid = (topv >= 0.0) & (lane < K)
    sqrt_scores = jnp.where(valid, jnp.sqrt(jnp.where(valid, topv, 1.0)), 0.0)
    labels = topc + 1

    area = jnp.maximum(x2 - x1, 0.0) * jnp.maximum(y2 - y1, 0.0)
    ltx = jnp.maximum(x1[:, :, None], x1[:, None, :])
    lty = jnp.maximum(y1[:, :, None], y1[:, None, :])
    rbx = jnp.minimum(x2[:, :, None], x2[:, None, :])
    rby = jnp.minimum(y2[:, :, None], y2[:, None, :])
    inter = jnp.maximum(rbx - ltx, 0.0) * jnp.maximum(rby - lty, 0.0)
    iou = inter / jnp.maximum(area[:, :, None] + area[:, None, :] - inter, 1e-9)
    iou_s[:] = iou

    keep0 = valid.astype(jnp.float32)

    def nms(i, keep):
        row = iou_s[:, pl.ds(i, 1), :].reshape(N, PAD)
        ki = jnp.max(jnp.where(lane == i, keep, 0.0), axis=1, keepdims=True)
        sup = (row > NMS_THR) & (ki > 0.5) & (lane > i)
        return jnp.where(sup, 0.0, keep)

    keep = lax.fori_loop(0, K, nms, keep0)
    keep_b = keep > 0.5

    boxes_ref[:, 0, :] = x1 * keep
    boxes_ref[:, 1, :] = y1 * keep
    boxes_ref[:, 2, :] = x2 * keep
    boxes_ref[:, 3, :] = y2 * keep
    scores_ref[:] = sqrt_scores * keep
    labels_ref[:] = jnp.where(keep_b, labels, 0)
    keep_ref[:] = keep_b.astype(jnp.int32)


@jax.jit
def kernel(locations, box_cls, box_regression, centerness, size):
    f32, i32 = jnp.float32, jnp.int32

    locmax = pl.pallas_call(
        _k1_body,
        grid=(N, HB),
        in_specs=[
            pl.BlockSpec((1, C, Hb, W), lambda n, j: (n, 0, j, 0)),
            pl.BlockSpec((1, 1, Hb, W), lambda n, j: (n, 0, j, 0)),
        ],
        out_specs=pl.BlockSpec((1, 1, BLK), lambda n, j: (n * HB + j, 0, 0)),
        out_shape=jax.ShapeDtypeStruct((N * HB, 1, BLK), f32),
    )(box_cls, centerness)
    locmax = locmax.reshape(N, HW)

    selhw = pl.pallas_call(
        _k2_body,
        out_shape=jax.ShapeDtypeStruct((N, PAD), i32),
    )(locmax)

    sco, reg_s, loc_s = pl.pallas_call(
        _k3_body,
        grid=(N, HB),
        in_specs=[
            pl.BlockSpec((1, C, Hb, W), lambda n, j: (n, 0, j, 0)),
            pl.BlockSpec((1, 1, Hb, W), lambda n, j: (n, 0, j, 0)),
            pl.BlockSpec((1, 4, Hb, W), lambda n, j: (n, 0, j, 0)),
            pl.BlockSpec((BLK, 2), lambda n, j: (j, 0)),
            pl.BlockSpec((1, 1, PAD), lambda n, j: (n, 0, 0)),
        ],
        out_specs=[
            pl.BlockSpec((1, C, PAD), lambda n, j: (n, 0, 0)),
            pl.BlockSpec((1, 4, PAD), lambda n, j: (n, 0, 0)),
            pl.BlockSpec((1, 2, PAD), lambda n, j: (n, 0, 0)),
        ],
        out_shape=[
            jax.ShapeDtypeStruct((N, C, PAD), f32),
            jax.ShapeDtypeStruct((N, 4, PAD), f32),
            jax.ShapeDtypeStruct((N, 2, PAD), f32),
        ],
    )(box_cls, centerness, box_regression, locations,
      selhw.reshape(N, 1, PAD))

    boxes_t, scores_p, labels_p, keep_p = pl.pallas_call(
        _k4_body,
        in_specs=[
            pl.BlockSpec((N, C, PAD), lambda: (0, 0, 0)),
            pl.BlockSpec((N, PAD), lambda: (0, 0)),
            pl.BlockSpec((N, 4, PAD), lambda: (0, 0, 0)),
            pl.BlockSpec((N, 2, PAD), lambda: (0, 0, 0)),
            pl.BlockSpec((1, 2), lambda: (0, 0)),
        ],
        out_specs=[
            pl.BlockSpec((N, 4, PAD), lambda: (0, 0, 0)),
            pl.BlockSpec((N, PAD), lambda: (0, 0)),
            pl.BlockSpec((N, PAD), lambda: (0, 0)),
            pl.BlockSpec((N, PAD), lambda: (0, 0)),
        ],
        out_shape=[
            jax.ShapeDtypeStruct((N, 4, PAD), f32),
            jax.ShapeDtypeStruct((N, PAD), f32),
            jax.ShapeDtypeStruct((N, PAD), i32),
            jax.ShapeDtypeStruct((N, PAD), i32),
        ],
        scratch_shapes=[pltpu.VMEM((N, PAD, PAD), f32)],
    )(sco, selhw, reg_s, loc_s, size.reshape(1, 2))

    boxes = jnp.moveaxis(boxes_t, 1, 2)[:, :K, :]
    return (boxes, scores_p[:, :K], labels_p[:, :K],
            keep_p[:, :K].astype(jnp.bool_))


# speedup vs baseline: 12.3359x; 1.1654x over previous
import functools

import jax
import jax.numpy as jnp
from jax import lax
from jax.experimental import pallas as pl
from jax.experimental.pallas import tpu as pltpu
from jax.experimental.pallas import tpu_sc as plsc

N, C, H, W = 4, 80, 160, 160
HW = H * W
HB = 10
Hb = H // HB
BLK = Hb * W
K = 100
THR = 0.2
NMS_THR = 0.6
PAD = 128
D = 128
NROW = N * PAD
BIG = 1 << 30

_DOT = dict(precision=lax.Precision.HIGHEST, preferred_element_type=jnp.float32)


def _sigmoid(x):
    return 1.0 / (1.0 + jnp.exp(-x))


def _k1_body(cls_ref, ctr_ref, reg_ref, loc_ref, locmax_ref, tab_ref):
    cls = cls_ref[0].reshape(C, BLK)
    ctr = ctr_ref[0, 0].reshape(1, BLK)
    rawmax = jnp.max(cls, axis=0, keepdims=True)
    sm = _sigmoid(rawmax)
    sc = _sigmoid(ctr)
    locmax_ref[0, 0, :] = jnp.where(sm > THR, sm * sc, -1.0)[0]
    big = jnp.concatenate(
        [cls, reg_ref[0].reshape(4, BLK), loc_ref[:].T, ctr,
         jnp.zeros((D - C - 7, BLK), jnp.float32)], axis=0)
    tab_ref[0] = big.T


def _k2_body(locmax_ref, selhw_ref, gidx_ref):
    x = locmax_ref[:]
    hw_iota = lax.broadcasted_iota(jnp.int32, (N, HW), 1)
    lane = lax.broadcasted_iota(jnp.int32, (N, PAD), 1)

    def body(i, carry):
        x, sel = carry
        m = jnp.max(x, axis=1, keepdims=True)
        win = jnp.min(jnp.where(x == m, hw_iota, BIG), axis=1, keepdims=True)
        sel = jnp.where(lane == i, win, sel)
        x = jnp.where(hw_iota == win, -2.0, x)
        return x, sel

    _, sel = lax.fori_loop(0, K, body, (x, jnp.full((N, PAD), -1, jnp.int32)))
    selhw_ref[:] = sel
    n_iota = lax.broadcasted_iota(jnp.int32, (N, PAD), 0)
    gidx_ref[:] = jnp.maximum(sel, 0) + n_iota * HW


def _make_sc_gather():
    info = plsc.get_sparse_core_info()
    nw = info.num_cores * info.num_subcores
    bpw = NROW // nw
    mesh = plsc.VectorSubcoreMesh(core_axis_name="c", subcore_axis_name="s")

    @functools.partial(
        pl.kernel, mesh=mesh,
        out_type=jax.ShapeDtypeStruct((NROW, D), jnp.float32),
        scratch_types=[
            pltpu.VMEM((bpw,), jnp.int32),
            pltpu.VMEM((bpw, D), jnp.float32),
            pltpu.SemaphoreType.DMA,
        ],
    )
    def gat(tab_hbm, idx_hbm, out_hbm, idx_v, rows_v, sem):
        wid = lax.axis_index("s") * info.num_cores + lax.axis_index("c")
        base = wid * bpw
        pltpu.sync_copy(idx_hbm.at[pl.ds(base, bpw)], idx_v)
        pltpu.async_copy(tab_hbm.at[idx_v], rows_v, sem).wait()
        pltpu.sync_copy(rows_v, out_hbm.at[pl.ds(base, bpw)])

    return gat


def _sc_gather(table, gidx):
    return _make_sc_gather()(table, gidx)


def _k4_body(rows_ref, selhw_ref, size_ref,
             boxes_ref, scores_ref, labels_ref, keep_ref, iou_s):
    rows = rows_ref[:]
    raw = rows[:, :, 0:C]
    sigc = _sigmoid(raw)
    sct = _sigmoid(rows[:, :, C + 6:C + 7])
    x = jnp.where(sigc > THR, sigc * sct, -1.0)
    selhw = selhw_ref[:]
    c_iota = lax.broadcasted_iota(jnp.int32, (N, PAD, C), 2)
    r_iota = lax.broadcasted_iota(jnp.int32, (N, PAD, C), 1)
    elem_idx = selhw[:, :, None] * C + c_iota
    lane = lax.broadcasted_iota(jnp.int32, (N, PAD), 1)

    x = jnp.where(selhw[:, :, None] < 0, -jnp.inf, x)

    def extract(i, carry):
        x, topv, topr, topc = carry
        m = jnp.max(jnp.max(x, axis=2), axis=1)
        mb = m[:, None, None]
        cand = x == mb
        wi = jnp.min(jnp.min(jnp.where(cand, elem_idx, BIG), axis=2), axis=1)
        wib = wi[:, None, None]
        hit = cand & (elem_idx == wib)
        wr = jnp.min(jnp.min(jnp.where(hit, r_iota, BIG), axis=2), axis=1)
        wc = wi % C
        eq = lane == i
        topv = jnp.where(eq, m[:, None], topv)
        topr = jnp.where(eq, wr[:, None], topr)
        topc = jnp.where(eq, wc[:, None], topc)
        x = jnp.where(elem_idx == wib, -jnp.inf, x)
        return x, topv, topr, topc

    init = (x, jnp.zeros((N, PAD), jnp.float32),
            jnp.zeros((N, PAD), jnp.int32), jnp.zeros((N, PAD), jnp.int32))
    _, topv, topr, topc = lax.fori_loop(0, K, extract, init)

    sds = []
    for n in range(N):
        ohn = (topr[n][:, None] ==
               lax.broadcasted_iota(jnp.int32, (PAD, PAD), 1)).astype(jnp.float32)
        data = rows[n, :, C:C + 6]
        sds.append(lax.dot_general(data, ohn, (((0,), (1,)), ((), ())), **_DOT))
    sd = jnp.stack(sds, axis=0)

    lx, ly = sd[:, 4, :], sd[:, 5, :]
    x1 = lx - sd[:, 0, :]
    y1 = ly - sd[:, 1, :]
    x2 = lx + sd[:, 2, :]
    y2 = ly + sd[:, 3, :]
    h_img = size_ref[0, 0].astype(jnp.float32)
    w_img = size_ref[0, 1].astype(jnp.float32)
    x1 = jnp.clip(x1, 0.0, w_img - 1.0)
    x2 = jnp.clip(x2, 0.0, w_img - 1.0)
    y1 = jnp.clip(y1, 0.0, h_img - 1.0)
    y2 = jnp.clip(y2, 0.0, h_img - 1.0)

    valid = (topv >= 0.0) & (lane < K)
    sqrt_scores = jnp.where(valid, jnp.sqrt(jnp.where(valid, topv, 1.0)), 0.0)
    labels = topc + 1

    area = jnp.maximum(x2 - x1, 0.0) * jnp.maximum(y2 - y1, 0.0)
    ltx = jnp.maximum(x1[:, :, None], x1[:, None, :])
    lty = jnp.maximum(y1[:, :, None], y1[:, None, :])
    rbx = jnp.minimum(x2[:, :, None], x2[:, None, :])
    rby = jnp.minimum(y2[:, :, None], y2[:, None, :])
    inter = jnp.maximum(rbx - ltx, 0.0) * jnp.maximum(rby - lty, 0.0)
    iou = inter / jnp.maximum(area[:, :, None] + area[:, None, :] - inter, 1e-9)
    iou_s[:] = iou

    keep0 = valid.astype(jnp.float32)

    def nms(i, keep):
        row = iou_s[:, pl.ds(i, 1), :].reshape(N, PAD)
        ki = jnp.max(jnp.where(lane == i, keep, 0.0), axis=1, keepdims=True)
        sup = (row > NMS_THR) & (ki > 0.5) & (lane > i)
        return jnp.where(sup, 0.0, keep)

    keep = lax.fori_loop(0, K, nms, keep0)
    keep_b = keep > 0.5

    boxes_ref[:, 0, :] = x1 * keep
    boxes_ref[:, 1, :] = y1 * keep
    boxes_ref[:, 2, :] = x2 * keep
    boxes_ref[:, 3, :] = y2 * keep
    scores_ref[:] = sqrt_scores * keep
    labels_ref[:] = jnp.where(keep_b, labels, 0)
    keep_ref[:] = keep_b.astype(jnp.int32)


@jax.jit
def kernel(locations, box_cls, box_regression, centerness, size):
    f32, i32 = jnp.float32, jnp.int32

    locmax, table = pl.pallas_call(
        _k1_body,
        grid=(N, HB),
        in_specs=[
            pl.BlockSpec((1, C, Hb, W), lambda n, j: (n, 0, j, 0)),
            pl.BlockSpec((1, 1, Hb, W), lambda n, j: (n, 0, j, 0)),
            pl.BlockSpec((1, 4, Hb, W), lambda n, j: (n, 0, j, 0)),
            pl.BlockSpec((BLK, 2), lambda n, j: (j, 0)),
        ],
        out_specs=[
            pl.BlockSpec((1, 1, BLK), lambda n, j: (n * HB + j, 0, 0)),
            pl.BlockSpec((1, BLK, D), lambda n, j: (n * HB + j, 0, 0)),
        ],
        out_shape=[
            jax.ShapeDtypeStruct((N * HB, 1, BLK), f32),
            jax.ShapeDtypeStruct((N * HB, BLK, D), f32),
        ],
    )(box_cls, centerness, box_regression, locations)

    selhw, gidx = pl.pallas_call(
        _k2_body,
        out_shape=[jax.ShapeDtypeStruct((N, PAD), i32),
                   jax.ShapeDtypeStruct((N, PAD), i32)],
    )(locmax.reshape(N, HW))

    rows = _sc_gather(table.reshape(N * HW, D), gidx.reshape(NROW))

    boxes_t, scores_p, labels_p, keep_p = pl.pallas_call(
        _k4_body,
        in_specs=[
            pl.BlockSpec((N, PAD, D), lambda: (0, 0, 0)),
            pl.BlockSpec((N, PAD), lambda: (0, 0)),
            pl.BlockSpec((1, 2), lambda: (0, 0)),
        ],
        out_specs=[
            pl.BlockSpec((N, 4, PAD), lambda: (0, 0, 0)),
            pl.BlockSpec((N, PAD), lambda: (0, 0)),
            pl.BlockSpec((N, PAD), lambda: (0, 0)),
            pl.BlockSpec((N, PAD), lambda: (0, 0)),
        ],
        out_shape=[
            jax.ShapeDtypeStruct((N, 4, PAD), f32),
            jax.ShapeDtypeStruct((N, PAD), f32),
            jax.ShapeDtypeStruct((N, PAD), i32),
            jax.ShapeDtypeStruct((N, PAD), i32),
        ],
        scratch_shapes=[pltpu.VMEM((N, PAD, PAD), f32)],
    )(rows.reshape(N, PAD, D), selhw, size.reshape(1, 2))

    boxes = jnp.moveaxis(boxes_t, 1, 2)[:, :K, :]
    return (boxes, scores_p[:, :K], labels_p[:, :K],
            keep_p[:, :K].astype(jnp.bool_))
